# baseline (speedup 1.0000x reference)
import functools

import jax
import jax.numpy as jnp
from jax import lax
from jax.experimental import pallas as pl
from jax.experimental.pallas import tpu as pltpu
from jax.experimental.pallas import tpu_sc as plsc

NC = 2
NS = 16
NW = NC * NS
LANES = 16

N = 10000
D = 128
E = 320000
EPW = E // NW
CHUNK = 80
NCHUNK = EPW // CHUNK
RPT = N // NS

BLK = 1000


def _vsm():
    return plsc.VectorSubcoreMesh(core_axis_name="c", subcore_axis_name="s")


def _sc_degree(ei4):
    @functools.partial(
        pl.kernel,
        out_type=jax.ShapeDtypeStruct((NC, N, LANES), jnp.float32),
        mesh=_vsm(),
        compiler_params=pltpu.CompilerParams(use_tc_tiling_on_sc=False),
        scratch_types=[
            pltpu.VMEM((NCHUNK, CHUNK), jnp.int32),
            pltpu.VMEM((CHUNK, LANES), jnp.float32),
            pltpu.VMEM((RPT, LANES), jnp.float32),
            pltpu.VMEM_SHARED((N, LANES), jnp.float32),
        ],
    )
    def deg_kernel(ei_hbm, out_hbm, idx_v, ones_v, zbuf, acc):
        cid = lax.axis_index("c")
        sid = lax.axis_index("s")
        wid = cid * NS + sid

        def fill_row(r, _):
            ones_v[r, :] = jnp.ones((LANES,), jnp.float32)
            return 0

        lax.fori_loop(0, CHUNK, fill_row, 0)

        def zfill_row(r, _):
            zbuf[r, :] = jnp.zeros((LANES,), jnp.float32)
            return 0

        lax.fori_loop(0, RPT, zfill_row, 0)

        pltpu.sync_copy(zbuf, acc.at[pl.ds(sid * RPT, RPT)])
        plsc.subcore_barrier()

        pltpu.sync_copy(ei_hbm.at[1, wid], idx_v)

        def body(c, _):
            pltpu.sync_copy(ones_v, acc.at[idx_v.at[c]], add=True)
            return 0

        lax.fori_loop(0, NCHUNK, body, 0)
        plsc.subcore_barrier()

        pltpu.sync_copy(
            acc.at[pl.ds(sid * RPT, RPT)],
            out_hbm.at[cid, pl.ds(sid * RPT, RPT)],
        )

    return deg_kernel(ei4)


def _sc_aggregate(hs, ei4):
    @functools.partial(
        pl.kernel,
        out_type=jax.ShapeDtypeStruct((NC, N, D), jnp.float32),
        mesh=_vsm(),
        compiler_params=pltpu.CompilerParams(use_tc_tiling_on_sc=False),
        scratch_types=[
            pltpu.VMEM((NCHUNK, CHUNK), jnp.int32),
            pltpu.VMEM((NCHUNK, CHUNK), jnp.int32),
            pltpu.VMEM((CHUNK, D), jnp.float32),
            pltpu.VMEM((CHUNK, D), jnp.float32),
            pltpu.VMEM_SHARED((N, D), jnp.float32),
            pltpu.SemaphoreType.DMA,
            pltpu.SemaphoreType.DMA,
        ],
    )
    def agg_kernel(hs_hbm, ei_hbm, out_hbm,
                   src_v, dst_v, buf0, buf1, acc, sem0, sem1):
        cid = lax.axis_index("c")
        sid = lax.axis_index("s")
        wid = cid * NS + sid

        def zfill_row(r, _):
            for c8 in range(D // LANES):
                buf0[r, pl.ds(c8 * LANES, LANES)] = jnp.zeros(
                    (LANES,), jnp.float32)
            return 0

        lax.fori_loop(0, CHUNK, zfill_row, 0)

        for z in range(RPT // CHUNK):
            pltpu.sync_copy(buf0, acc.at[pl.ds(sid * RPT + z * CHUNK, CHUNK)])
        pltpu.sync_copy(
            buf0.at[pl.ds(0, RPT % CHUNK)],
            acc.at[pl.ds(sid * RPT + (RPT // CHUNK) * CHUNK, RPT % CHUNK)],
        )
        plsc.subcore_barrier()

        pltpu.sync_copy(ei_hbm.at[0, wid], src_v)
        pltpu.sync_copy(ei_hbm.at[1, wid], dst_v)

        pltpu.async_copy(hs_hbm.at[src_v.at[0]], buf0, sem0)

        def body(g, _):
            c0 = 2 * g
            c1 = 2 * g + 1
            pltpu.async_copy(hs_hbm.at[src_v.at[c1]], buf1, sem1)
            pltpu.make_async_copy(hs_hbm.at[src_v.at[c0]], buf0, sem0).wait()
            pltpu.sync_copy(buf0, acc.at[dst_v.at[c0]], add=True)

            @pl.when(c1 + 1 < NCHUNK)
            def _():
                pltpu.async_copy(hs_hbm.at[src_v.at[c1 + 1]], buf0, sem0)

            pltpu.make_async_copy(hs_hbm.at[src_v.at[c1]], buf1, sem1).wait()
            pltpu.sync_copy(buf1, acc.at[dst_v.at[c1]], add=True)
            return 0

        lax.fori_loop(0, NCHUNK // 2, body, 0)
        if NCHUNK % 2:
            c_last = NCHUNK - 1
            pltpu.make_async_copy(
                hs_hbm.at[src_v.at[c_last]], buf0, sem0).wait()
            pltpu.sync_copy(buf0, acc.at[dst_v.at[c_last]], add=True)
        plsc.subcore_barrier()

        pltpu.sync_copy(
            acc.at[pl.ds(sid * RPT, RPT)],
            out_hbm.at[cid, pl.ds(sid * RPT, RPT)],
        )

    return agg_kernel(hs, ei4)


def _dinv_block(degp_ref):
    deg = degp_ref[0, :, 0] + degp_ref[1, :, 0] + 1.0
    return lax.rsqrt(deg)[:, None]


def _tc_matmul_plain(x, W):
    def body(x_ref, w_ref, o_ref):
        o_ref[...] = jnp.dot(
            x_ref[...], w_ref[...], preferred_element_type=jnp.float32)

    return pl.pallas_call(
        body,
        grid=(N // BLK,),
        in_specs=[
            pl.BlockSpec((BLK, D), lambda i: (i, 0)),
            pl.BlockSpec((D, D), lambda i: (0, 0)),
        ],
        out_specs=pl.BlockSpec((BLK, D), lambda i: (i, 0)),
        out_shape=jax.ShapeDtypeStruct((N, D), jnp.float32),
    )(x, W)


def _tc_scale(degp, u):
    def body(degp_ref, u_ref, o_ref):
        o_ref[...] = _dinv_block(degp_ref) * u_ref[...]

    return pl.pallas_call(
        body,
        grid=(N // BLK,),
        in_specs=[
            pl.BlockSpec((NC, BLK, LANES), lambda i: (0, i, 0)),
            pl.BlockSpec((BLK, D), lambda i: (i, 0)),
        ],
        out_specs=pl.BlockSpec((BLK, D), lambda i: (i, 0)),
        out_shape=jax.ShapeDtypeStruct((N, D), jnp.float32),
    )(degp, u)


def _tc_mid(degp, p, hs1, b1, W2):
    def body(degp_ref, p_ref, hs_ref, b_ref, w_ref, o_ref):
        dinv = _dinv_block(degp_ref)
        z = dinv * (p_ref[0] + p_ref[1] + hs_ref[...]) + b_ref[...]
        a = jnp.maximum(z, 0.0)
        o_ref[...] = dinv * jnp.dot(
            a, w_ref[...], preferred_element_type=jnp.float32)

    return pl.pallas_call(
        body,
        grid=(N // BLK,),
        in_specs=[
            pl.BlockSpec((NC, BLK, LANES), lambda i: (0, i, 0)),
            pl.BlockSpec((NC, BLK, D), lambda i: (0, i, 0)),
            pl.BlockSpec((BLK, D), lambda i: (i, 0)),
            pl.BlockSpec((1, D), lambda i: (0, 0)),
            pl.BlockSpec((D, D), lambda i: (0, 0)),
        ],
        out_specs=pl.BlockSpec((BLK, D), lambda i: (i, 0)),
        out_shape=jax.ShapeDtypeStruct((N, D), jnp.float32),
    )(degp, p, hs1, b1, W2)


def _tc_final(degp, q, hs2, b2):
    def body(degp_ref, q_ref, hs_ref, b_ref, o_ref):
        dinv = _dinv_block(degp_ref)
        o_ref[...] = dinv * (q_ref[0] + q_ref[1] + hs_ref[...]) + b_ref[...]

    return pl.pallas_call(
        body,
        grid=(N // BLK,),
        in_specs=[
            pl.BlockSpec((NC, BLK, LANES), lambda i: (0, i, 0)),
            pl.BlockSpec((NC, BLK, D), lambda i: (0, i, 0)),
            pl.BlockSpec((BLK, D), lambda i: (i, 0)),
            pl.BlockSpec((1, D), lambda i: (0, 0)),
        ],
        out_specs=pl.BlockSpec((BLK, D), lambda i: (i, 0)),
        out_shape=jax.ShapeDtypeStruct((N, D), jnp.float32),
    )(degp, q, hs2, b2)


def kernel(x, edge_index, W1, b1, W2, b2):
    ei4 = edge_index.astype(jnp.int32).reshape(2, NW, NCHUNK, CHUNK)
    b1r = b1.reshape(1, D)
    b2r = b2.reshape(1, D)

    u1 = _tc_matmul_plain(x, W1)
    degp = _sc_degree(ei4)
    hs1 = _tc_scale(degp, u1)
    p = _sc_aggregate(hs1, ei4)
    hs2 = _tc_mid(degp, p, hs1, b1r, W2)
    q = _sc_aggregate(hs2, ei4)
    out = _tc_final(degp, q, hs2, b2r)
    return out


# speedup vs baseline: 1.0245x; 1.0245x over previous
import functools

import jax
import jax.numpy as jnp
from jax import lax
from jax.experimental import pallas as pl
from jax.experimental.pallas import tpu as pltpu
from jax.experimental.pallas import tpu_sc as plsc

NC = 2
NS = 16
NW = NC * NS
LANES = 16

N = 10000
D = 128
E = 320000
EPW = E // NW
CHUNK = 100
NCHUNK = EPW // CHUNK
RPT = N // NS

BLK = 1000


def _vsm():
    return plsc.VectorSubcoreMesh(core_axis_name="c", subcore_axis_name="s")


def _sc_degree(ei4):
    @functools.partial(
        pl.kernel,
        out_type=jax.ShapeDtypeStruct((NC, N, LANES), jnp.float32),
        mesh=_vsm(),
        compiler_params=pltpu.CompilerParams(use_tc_tiling_on_sc=False),
        scratch_types=[
            pltpu.VMEM((NCHUNK, CHUNK), jnp.int32),
            pltpu.VMEM((CHUNK, LANES), jnp.float32),
            pltpu.VMEM((RPT, LANES), jnp.float32),
            pltpu.VMEM_SHARED((N, LANES), jnp.float32),
        ],
    )
    def deg_kernel(ei_hbm, out_hbm, idx_v, ones_v, zbuf, acc):
        cid = lax.axis_index("c")
        sid = lax.axis_index("s")
        wid = cid * NS + sid

        def fill_row(r, _):
            ones_v[r, :] = jnp.ones((LANES,), jnp.float32)
            return 0

        lax.fori_loop(0, CHUNK, fill_row, 0)

        def zfill_row(r, _):
            zbuf[r, :] = jnp.zeros((LANES,), jnp.float32)
            return 0

        lax.fori_loop(0, RPT, zfill_row, 0)

        pltpu.sync_copy(zbuf, acc.at[pl.ds(sid * RPT, RPT)])
        plsc.subcore_barrier()

        pltpu.sync_copy(ei_hbm.at[1, wid], idx_v)

        def body(c, _):
            pltpu.sync_copy(ones_v, acc.at[idx_v.at[c]], add=True)
            return 0

        lax.fori_loop(0, NCHUNK, body, 0)
        plsc.subcore_barrier()

        pltpu.sync_copy(
            acc.at[pl.ds(sid * RPT, RPT)],
            out_hbm.at[cid, pl.ds(sid * RPT, RPT)],
        )

    return deg_kernel(ei4)


def _sc_aggregate(hs, ei4):
    @functools.partial(
        pl.kernel,
        out_type=jax.ShapeDtypeStruct((NC, N, D), jnp.float32),
        mesh=_vsm(),
        compiler_params=pltpu.CompilerParams(use_tc_tiling_on_sc=False),
        scratch_types=[
            pltpu.VMEM((NCHUNK, CHUNK), jnp.int32),
            pltpu.VMEM((NCHUNK, CHUNK), jnp.int32),
            pltpu.VMEM((CHUNK, D), jnp.float32),
            pltpu.VMEM((CHUNK, D), jnp.float32),
            pltpu.VMEM_SHARED((N, D), jnp.float32),
            pltpu.SemaphoreType.DMA,
            pltpu.SemaphoreType.DMA,
        ],
    )
    def agg_kernel(hs_hbm, ei_hbm, out_hbm,
                   src_v, dst_v, buf0, buf1, acc, sem0, sem1):
        cid = lax.axis_index("c")
        sid = lax.axis_index("s")
        wid = cid * NS + sid

        def zfill_row(r, _):
            for c8 in range(D // LANES):
                buf0[r, pl.ds(c8 * LANES, LANES)] = jnp.zeros(
                    (LANES,), jnp.float32)
            return 0

        lax.fori_loop(0, CHUNK, zfill_row, 0)

        for z in range(RPT // CHUNK):
            pltpu.sync_copy(buf0, acc.at[pl.ds(sid * RPT + z * CHUNK, CHUNK)])
        pltpu.sync_copy(
            buf0.at[pl.ds(0, RPT % CHUNK)],
            acc.at[pl.ds(sid * RPT + (RPT // CHUNK) * CHUNK, RPT % CHUNK)],
        )
        plsc.subcore_barrier()

        pltpu.sync_copy(ei_hbm.at[0, wid], src_v)
        pltpu.sync_copy(ei_hbm.at[1, wid], dst_v)

        pltpu.async_copy(hs_hbm.at[src_v.at[0]], buf0, sem0)

        def body(g, _):
            c0 = 2 * g
            c1 = 2 * g + 1
            pltpu.async_copy(hs_hbm.at[src_v.at[c1]], buf1, sem1)
            pltpu.make_async_copy(hs_hbm.at[src_v.at[c0]], buf0, sem0).wait()
            pltpu.sync_copy(buf0, acc.at[dst_v.at[c0]], add=True)

            @pl.when(c1 + 1 < NCHUNK)
            def _():
                pltpu.async_copy(hs_hbm.at[src_v.at[c1 + 1]], buf0, sem0)

            pltpu.make_async_copy(hs_hbm.at[src_v.at[c1]], buf1, sem1).wait()
            pltpu.sync_copy(buf1, acc.at[dst_v.at[c1]], add=True)
            return 0

        lax.fori_loop(0, NCHUNK // 2, body, 0)
        if NCHUNK % 2:
            c_last = NCHUNK - 1
            pltpu.make_async_copy(
                hs_hbm.at[src_v.at[c_last]], buf0, sem0).wait()
            pltpu.sync_copy(buf0, acc.at[dst_v.at[c_last]], add=True)
        plsc.subcore_barrier()

        pltpu.sync_copy(
            acc.at[pl.ds(sid * RPT, RPT)],
            out_hbm.at[cid, pl.ds(sid * RPT, RPT)],
        )

    return agg_kernel(hs, ei4)


def _dinv_block(degp_ref):
    deg = degp_ref[0, :, 0] + degp_ref[1, :, 0] + 1.0
    return lax.rsqrt(deg)[:, None]


def _tc_matmul_plain(x, W):
    def body(x_ref, w_ref, o_ref):
        o_ref[...] = jnp.dot(
            x_ref[...], w_ref[...], preferred_element_type=jnp.float32)

    return pl.pallas_call(
        body,
        grid=(N // BLK,),
        in_specs=[
            pl.BlockSpec((BLK, D), lambda i: (i, 0)),
            pl.BlockSpec((D, D), lambda i: (0, 0)),
        ],
        out_specs=pl.BlockSpec((BLK, D), lambda i: (i, 0)),
        out_shape=jax.ShapeDtypeStruct((N, D), jnp.float32),
    )(x, W)


def _tc_scale(degp, u):
    def body(degp_ref, u_ref, o_ref):
        o_ref[...] = _dinv_block(degp_ref) * u_ref[...]

    return pl.pallas_call(
        body,
        grid=(N // BLK,),
        in_specs=[
            pl.BlockSpec((NC, BLK, LANES), lambda i: (0, i, 0)),
            pl.BlockSpec((BLK, D), lambda i: (i, 0)),
        ],
        out_specs=pl.BlockSpec((BLK, D), lambda i: (i, 0)),
        out_shape=jax.ShapeDtypeStruct((N, D), jnp.float32),
    )(degp, u)


def _tc_mid(degp, p, hs1, b1, W2):
    def body(degp_ref, p_ref, hs_ref, b_ref, w_ref, o_ref):
        dinv = _dinv_block(degp_ref)
        z = dinv * (p_ref[0] + p_ref[1] + hs_ref[...]) + b_ref[...]
        a = jnp.maximum(z, 0.0)
        o_ref[...] = dinv * jnp.dot(
            a, w_ref[...], preferred_element_type=jnp.float32)

    return pl.pallas_call(
        body,
        grid=(N // BLK,),
        in_specs=[
            pl.BlockSpec((NC, BLK, LANES), lambda i: (0, i, 0)),
            pl.BlockSpec((NC, BLK, D), lambda i: (0, i, 0)),
            pl.BlockSpec((BLK, D), lambda i: (i, 0)),
            pl.BlockSpec((1, D), lambda i: (0, 0)),
            pl.BlockSpec((D, D), lambda i: (0, 0)),
        ],
        out_specs=pl.BlockSpec((BLK, D), lambda i: (i, 0)),
        out_shape=jax.ShapeDtypeStruct((N, D), jnp.float32),
    )(degp, p, hs1, b1, W2)


def _tc_final(degp, q, hs2, b2):
    def body(degp_ref, q_ref, hs_ref, b_ref, o_ref):
        dinv = _dinv_block(degp_ref)
        o_ref[...] = dinv * (q_ref[0] + q_ref[1] + hs_ref[...]) + b_ref[...]

    return pl.pallas_call(
        body,
        grid=(N // BLK,),
        in_specs=[
            pl.BlockSpec((NC, BLK, LANES), lambda i: (0, i, 0)),
            pl.BlockSpec((NC, BLK, D), lambda i: (0, i, 0)),
            pl.BlockSpec((BLK, D), lambda i: (i, 0)),
            pl.BlockSpec((1, D), lambda i: (0, 0)),
        ],
        out_specs=pl.BlockSpec((BLK, D), lambda i: (i, 0)),
        out_shape=jax.ShapeDtypeStruct((N, D), jnp.float32),
    )(degp, q, hs2, b2)


def kernel(x, edge_index, W1, b1, W2, b2):
    ei4 = edge_index.astype(jnp.int32).reshape(2, NW, NCHUNK, CHUNK)
    b1r = b1.reshape(1, D)
    b2r = b2.reshape(1, D)

    u1 = _tc_matmul_plain(x, W1)
    degp = _sc_degree(ei4)
    hs1 = _tc_scale(degp, u1)
    p = _sc_aggregate(hs1, ei4)
    hs2 = _tc_mid(degp, p, hs1, b1r, W2)
    q = _sc_aggregate(hs2, ei4)
    out = _tc_final(degp, q, hs2, b2r)
    return out
